# Initial kernel scaffold; baseline (speedup 1.0000x reference)
import jax
import jax.numpy as jnp
from jax.experimental import pallas as pl


def kernel(hidden_states, W_route, W_noise):
    raise NotImplementedError("write your pallas kernel here")


# speedup vs baseline: 5.0561x; 5.0561x over previous
import functools

import jax
import jax.numpy as jnp
from jax.experimental import pallas as pl

TOP_K = 8
N_EXPERTS = 64
TILE_N = 512


def _router_body(x_ref, wt_ref, out_ref, idx_ref):
    x = x_ref[...]
    wt = wt_ref[...]
    logits = jnp.dot(x, wt, preferred_element_type=jnp.float32)
    tile_n = logits.shape[0]
    col = jax.lax.broadcasted_iota(jnp.int32, logits.shape, 1)
    lane_k = jax.lax.broadcasted_iota(jnp.int32, (tile_n, TOP_K), 1)

    work = logits
    out_acc = jnp.zeros_like(logits)
    idx_acc = jnp.zeros((tile_n, TOP_K), jnp.int32)
    ssum = jnp.zeros((tile_n, 1), jnp.float32)
    m0 = None
    for j in range(TOP_K):
        m = jnp.max(work, axis=-1, keepdims=True)
        idx = jnp.min(
            jnp.where(work == m, col, N_EXPERTS), axis=-1, keepdims=True
        )
        if j == 0:
            m0 = m
        e = jnp.exp(m - m0)
        ssum = ssum + e
        onehot = col == idx
        out_acc = out_acc + jnp.where(onehot, e, 0.0)
        idx_acc = idx_acc + jnp.where(lane_k == j, idx, 0)
        work = jnp.where(onehot, -jnp.inf, work)

    out_ref[...] = out_acc / ssum
    idx_ref[...] = idx_acc


@jax.jit
def _router(hidden_states, wt):
    n, d = hidden_states.shape
    e = wt.shape[1]
    grid = (n // TILE_N,)
    return pl.pallas_call(
        _router_body,
        grid=grid,
        in_specs=[
            pl.BlockSpec((TILE_N, d), lambda i: (i, 0)),
            pl.BlockSpec((d, e), lambda i: (0, 0)),
        ],
        out_specs=[
            pl.BlockSpec((TILE_N, e), lambda i: (i, 0)),
            pl.BlockSpec((TILE_N, TOP_K), lambda i: (i, 0)),
        ],
        out_shape=[
            jax.ShapeDtypeStruct((n, e), jnp.float32),
            jax.ShapeDtypeStruct((n, TOP_K), jnp.int32),
        ],
    )(hidden_states, wt)


def kernel(hidden_states, W_route, W_noise):
    del W_noise
    router_output, indices = _router(hidden_states, W_route.T)
    return (router_output, indices)


# speedup vs baseline: 6.2372x; 1.2336x over previous
import functools

import jax
import jax.numpy as jnp
from jax.experimental import pallas as pl

TOP_K = 8
N_EXPERTS = 64
TILE_N = 512


def _router_body(x_ref, wt_ref, out_ref, idx_ref):
    x = x_ref[...]
    wt = wt_ref[...]
    logits = jnp.dot(x, wt, preferred_element_type=jnp.float32)
    lt = logits.T
    tile_n = lt.shape[1]
    erow = jax.lax.broadcasted_iota(jnp.int32, lt.shape, 0)
    krow = jax.lax.broadcasted_iota(jnp.int32, (TOP_K, tile_n), 0)

    work = lt
    out_t = jnp.zeros_like(lt)
    iacc = jnp.zeros((TOP_K, tile_n), jnp.int32)
    ssum = jnp.zeros((1, tile_n), jnp.float32)
    m0 = None
    for j in range(TOP_K):
        m = jnp.max(work, axis=0, keepdims=True)
        idx = jnp.min(
            jnp.where(work == m, erow, N_EXPERTS), axis=0, keepdims=True
        )
        if j == 0:
            m0 = m
        e = jnp.exp(m - m0)
        ssum = ssum + e
        onehot = erow == idx
        out_t = out_t + jnp.where(onehot, e, 0.0)
        iacc = iacc + jnp.where(krow == j, idx, 0)
        work = jnp.where(onehot, -jnp.inf, work)

    out_ref[...] = (out_t / ssum).T
    idx_ref[...] = iacc.T


@jax.jit
def _router(hidden_states, wt):
    n, d = hidden_states.shape
    e = wt.shape[1]
    grid = (n // TILE_N,)
    return pl.pallas_call(
        _router_body,
        grid=grid,
        in_specs=[
            pl.BlockSpec((TILE_N, d), lambda i: (i, 0)),
            pl.BlockSpec((d, e), lambda i: (0, 0)),
        ],
        out_specs=[
            pl.BlockSpec((TILE_N, e), lambda i: (i, 0)),
            pl.BlockSpec((TILE_N, TOP_K), lambda i: (i, 0)),
        ],
        out_shape=[
            jax.ShapeDtypeStruct((n, e), jnp.float32),
            jax.ShapeDtypeStruct((n, TOP_K), jnp.int32),
        ],
    )(hidden_states, wt)


def kernel(hidden_states, W_route, W_noise):
    del W_noise
    router_output, indices = _router(hidden_states, W_route.T)
    return (router_output, indices)


# speedup vs baseline: 6.6694x; 1.0693x over previous
import functools

import jax
import jax.numpy as jnp
from jax.experimental import pallas as pl

TOP_K = 8
N_EXPERTS = 64
TILE_N = 1024


def _router_body(x_ref, wt_ref, out_ref, idx_ref):
    x = x_ref[...]
    wt = wt_ref[...]
    logits = jnp.dot(x, wt, preferred_element_type=jnp.float32)
    lt = logits.T
    tile_n = lt.shape[1]
    erow = jax.lax.broadcasted_iota(jnp.int32, lt.shape, 0)
    krow = jax.lax.broadcasted_iota(jnp.int32, (TOP_K, tile_n), 0)

    work = lt
    out_t = jnp.zeros_like(lt)
    iacc = jnp.zeros((TOP_K, tile_n), jnp.int32)
    ssum = jnp.zeros((1, tile_n), jnp.float32)
    m0 = None
    for j in range(TOP_K):
        m = jnp.max(work, axis=0, keepdims=True)
        idx = jnp.min(
            jnp.where(work == m, erow, N_EXPERTS), axis=0, keepdims=True
        )
        if j == 0:
            m0 = m
        e = jnp.exp(m - m0)
        ssum = ssum + e
        onehot = erow == idx
        out_t = out_t + jnp.where(onehot, e, 0.0)
        iacc = iacc + jnp.where(krow == j, idx, 0)
        work = jnp.where(onehot, -jnp.inf, work)

    out_ref[...] = (out_t / ssum).T
    idx_ref[...] = iacc.T


@jax.jit
def _router(hidden_states, wt):
    n, d = hidden_states.shape
    e = wt.shape[1]
    grid = (n // TILE_N,)
    return pl.pallas_call(
        _router_body,
        grid=grid,
        in_specs=[
            pl.BlockSpec((TILE_N, d), lambda i: (i, 0)),
            pl.BlockSpec((d, e), lambda i: (0, 0)),
        ],
        out_specs=[
            pl.BlockSpec((TILE_N, e), lambda i: (i, 0)),
            pl.BlockSpec((TILE_N, TOP_K), lambda i: (i, 0)),
        ],
        out_shape=[
            jax.ShapeDtypeStruct((n, e), jnp.float32),
            jax.ShapeDtypeStruct((n, TOP_K), jnp.int32),
        ],
    )(hidden_states, wt)


def kernel(hidden_states, W_route, W_noise):
    del W_noise
    router_output, indices = _router(hidden_states, W_route.T)
    return (router_output, indices)


# speedup vs baseline: 6.6755x; 1.0009x over previous
import functools

import jax
import jax.numpy as jnp
from jax.experimental import pallas as pl
from jax.experimental.pallas import tpu as pltpu

TOP_K = 8
N_EXPERTS = 64
TILE_N = 1024


def _router_body(x_ref, wt_ref, out_ref, idx_ref):
    x = x_ref[...]
    wt = wt_ref[...]
    logits = jnp.dot(x, wt, preferred_element_type=jnp.float32)
    lt = logits.T
    tile_n = lt.shape[1]
    erow = jax.lax.broadcasted_iota(jnp.int32, lt.shape, 0)
    krow = jax.lax.broadcasted_iota(jnp.int32, (TOP_K, tile_n), 0)

    work = lt
    out_t = jnp.zeros_like(lt)
    iacc = jnp.zeros((TOP_K, tile_n), jnp.int32)
    ssum = jnp.zeros((1, tile_n), jnp.float32)
    m0 = None
    for j in range(TOP_K):
        m = jnp.max(work, axis=0, keepdims=True)
        idx = jnp.min(
            jnp.where(work == m, erow, N_EXPERTS), axis=0, keepdims=True
        )
        if j == 0:
            m0 = m
        e = jnp.exp(m - m0)
        ssum = ssum + e
        onehot = erow == idx
        out_t = out_t + jnp.where(onehot, e, 0.0)
        iacc = iacc + jnp.where(krow == j, idx, 0)
        work = jnp.where(onehot, -jnp.inf, work)

    out_ref[...] = (out_t / ssum).T
    idx_ref[...] = iacc.T


@jax.jit
def _router(hidden_states, wt):
    n, d = hidden_states.shape
    e = wt.shape[1]
    grid = (n // TILE_N,)
    return pl.pallas_call(
        _router_body,
        grid=grid,
        in_specs=[
            pl.BlockSpec((TILE_N, d), lambda i: (i, 0)),
            pl.BlockSpec((d, e), lambda i: (0, 0)),
        ],
        out_specs=[
            pl.BlockSpec((TILE_N, e), lambda i: (i, 0)),
            pl.BlockSpec((TILE_N, TOP_K), lambda i: (i, 0)),
        ],
        out_shape=[
            jax.ShapeDtypeStruct((n, e), jnp.float32),
            jax.ShapeDtypeStruct((n, TOP_K), jnp.int32),
        ],
        compiler_params=pltpu.CompilerParams(
            dimension_semantics=("parallel",),
        ),
    )(hidden_states, wt)


def kernel(hidden_states, W_route, W_noise):
    del W_noise
    router_output, indices = _router(hidden_states, W_route.T)
    return (router_output, indices)
